# baseline (speedup 1.0000x reference)
import dataclasses

import jax
import jax.numpy as jnp
from jax.experimental import pallas as pl
from jax.experimental.pallas import tpu as pltpu
from jax.experimental.pallas import tpu_sc as plsc

_RB = 8
_W = _RB * 50
_STARTS = (0, 104, 208, 312, _W)
_NC = len(_STARTS) - 1
_CMAX = max(b - a for a, b in zip(_STARTS[:-1], _STARTS[1:]))
_LANES = 16


def _compiler_params():
    cp = pltpu.CompilerParams()
    if "needs_layout_passes" in pltpu.CompilerParams.__dataclass_fields__:
        cp = dataclasses.replace(cp, needs_layout_passes=False)
    return cp


def kernel(embed_id, weight):
    B, S = embed_id.shape
    N = B * S
    V, D = weight.shape
    n_blocks = N // _W

    flat = embed_id.reshape(N)
    w_lines = jnp.concatenate([weight[0::4], weight[1::4], weight[2::4], weight[3::4]], axis=1)

    mesh = plsc.VectorSubcoreMesh(core_axis_name="core", subcore_axis_name="subcore")

    @pl.kernel(
        out_type=jax.ShapeDtypeStruct((B, S, D), weight.dtype),
        mesh=mesh,
        scratch_types=[
            pltpu.VMEM((_W,), jnp.int32),
            pltpu.VMEM((2, _CMAX, 4 * D), weight.dtype),
            pltpu.SemaphoreType.DMA,
            pltpu.SemaphoreType.DMA,
        ],
        compiler_params=_compiler_params(),
    )
    def gather_kernel(w_hbm, f_hbm, o_hbm, qb, buf, sem0, sem1):
        sems = (sem0, sem1)

        def body(f_vmem, o_vmem):
            for g in range(0, _W, _LANES):
                qb[pl.ds(g, _LANES)] = f_vmem[pl.ds(g, _LANES)] >> 2

            def fire(c):
                lo, hi = _STARTS[c], _STARTS[c + 1]
                return pltpu.async_copy(
                    w_hbm.at[qb.at[pl.ds(lo, hi - lo)]],
                    buf.at[c % 2, pl.ds(0, hi - lo)],
                    sems[c % 2],
                )

            def select(row, rv, k):
                c = next(i for i in range(_NC) if row < _STARTS[i + 1])
                brow = row - _STARTS[c]
                rr, j = divmod(row, 50)
                r = rv[k]
                o_vmem[rr, j, pl.ds(0, _LANES)] = buf[
                    c % 2, brow, pl.ds(r, _LANES)
                ]
                o_vmem[rr, j, pl.ds(_LANES, _LANES)] = buf[
                    c % 2, brow, pl.ds(r + _LANES, _LANES)
                ]

            handles = [fire(0)]
            for c in range(_NC):
                if c + 1 < _NC:
                    handles.append(fire(c + 1))
                handles[c].wait()
                lo, hi = _STARTS[c], _STARTS[c + 1]
                g0 = (lo // _LANES) * _LANES
                for g in range(g0, hi, _LANES):
                    rv = (f_vmem[pl.ds(g, _LANES)] & 3) << 5
                    for k in range(_LANES):
                        if lo <= g + k < hi:
                            select(g + k, rv, k)

        pltpu.emit_pipeline(
            body,
            grid=(n_blocks,),
            in_specs=[pl.BlockSpec((_W,), index_map=lambda i: (i,))],
            out_specs=[pl.BlockSpec((_RB, S, D), index_map=lambda i: (i, 0, 0))],
            core_axis_name=("core", "subcore"),
            dimension_semantics=(pltpu.PARALLEL,),
        )(f_hbm, o_hbm)

    return gather_kernel(w_lines, flat)


# speedup vs baseline: 4.2570x; 4.2570x over previous
import dataclasses

import jax
import jax.numpy as jnp
from jax.experimental import pallas as pl
from jax.experimental.pallas import tpu as pltpu
from jax.experimental.pallas import tpu_sc as plsc

_RB = 8
_W = _RB * 50
_STARTS = (0, 104, 208, 312, _W)
_NC = len(_STARTS) - 1
_CMAX = max(b - a for a, b in zip(_STARTS[:-1], _STARTS[1:]))
_LANES = 16


def _compiler_params():
    cp = pltpu.CompilerParams()
    if "needs_layout_passes" in pltpu.CompilerParams.__dataclass_fields__:
        cp = dataclasses.replace(cp, needs_layout_passes=False)
    return cp


def kernel(embed_id, weight):
    B, S = embed_id.shape
    N = B * S
    V, D = weight.shape
    n_blocks = N // _W

    flat = embed_id.reshape(N)
    one = (flat[0] * 0 + 1).astype(weight.dtype)
    w_lines = weight.reshape(V // 4, 4 * D) * one

    mesh = plsc.VectorSubcoreMesh(core_axis_name="core", subcore_axis_name="subcore")

    @pl.kernel(
        out_type=jax.ShapeDtypeStruct((B, S, D), weight.dtype),
        mesh=mesh,
        scratch_types=[
            pltpu.VMEM((_W,), jnp.int32),
            pltpu.VMEM((2, _CMAX, 4 * D), weight.dtype),
            pltpu.SemaphoreType.DMA,
            pltpu.SemaphoreType.DMA,
        ],
        compiler_params=_compiler_params(),
    )
    def gather_kernel(w_hbm, f_hbm, o_hbm, qb, buf, sem0, sem1):
        sems = (sem0, sem1)

        def body(f_vmem, o_vmem):
            for g in range(0, _W, _LANES):
                qb[pl.ds(g, _LANES)] = f_vmem[pl.ds(g, _LANES)] >> 2

            def fire(c):
                lo, hi = _STARTS[c], _STARTS[c + 1]
                return pltpu.async_copy(
                    w_hbm.at[qb.at[pl.ds(lo, hi - lo)]],
                    buf.at[c % 2, pl.ds(0, hi - lo)],
                    sems[c % 2],
                )

            def select(row, rv, k):
                c = next(i for i in range(_NC) if row < _STARTS[i + 1])
                brow = row - _STARTS[c]
                rr, j = divmod(row, 50)
                r = rv[k]
                o_vmem[rr, j, pl.ds(0, _LANES)] = buf[
                    c % 2, brow, pl.ds(r, _LANES)
                ]
                o_vmem[rr, j, pl.ds(_LANES, _LANES)] = buf[
                    c % 2, brow, pl.ds(r + _LANES, _LANES)
                ]

            handles = [fire(0)]
            for c in range(_NC):
                if c + 1 < _NC:
                    handles.append(fire(c + 1))
                handles[c].wait()
                lo, hi = _STARTS[c], _STARTS[c + 1]
                g0 = (lo // _LANES) * _LANES
                for g in range(g0, hi, _LANES):
                    rv = (f_vmem[pl.ds(g, _LANES)] & 3) << 5
                    for k in range(_LANES):
                        if lo <= g + k < hi:
                            select(g + k, rv, k)

        pltpu.emit_pipeline(
            body,
            grid=(n_blocks,),
            in_specs=[pl.BlockSpec((_W,), index_map=lambda i: (i,))],
            out_specs=[pl.BlockSpec((_RB, S, D), index_map=lambda i: (i, 0, 0))],
            core_axis_name=("core", "subcore"),
            dimension_semantics=(pltpu.PARALLEL,),
        )(f_hbm, o_hbm)

    return gather_kernel(w_lines, flat)
